# Initial kernel scaffold; baseline (speedup 1.0000x reference)
import jax
import jax.numpy as jnp
from jax.experimental import pallas as pl


def kernel(x, edge_index, edge_weight, W1, b1, W2, b2):
    raise NotImplementedError("write your pallas kernel here")


# speedup vs baseline: 7.9552x; 7.9552x over previous
import functools

import jax
import jax.numpy as jnp
from jax import lax
from jax.experimental import pallas as pl
from jax.experimental.pallas import tpu as pltpu
from jax.experimental.pallas import tpu_sc as plsc

NC = 2
NS = 16
LB = 80


def _make_deg_kernel(n, e):
    eb = e // (NC * NS)
    nb = eb // LB
    zr = 640
    mesh = plsc.VectorSubcoreMesh(core_axis_name="c", subcore_axis_name="s")

    @functools.partial(
        pl.kernel,
        out_type=jax.ShapeDtypeStruct((NC * n,), jnp.float32),
        mesh=mesh,
        scratch_types=[
            pltpu.VMEM((LB,), jnp.int32),
            pltpu.VMEM((LB,), jnp.float32),
            pltpu.VMEM((zr,), jnp.float32),
            pltpu.VMEM_SHARED((n,), jnp.float32),
        ],
    )
    def deg_kernel(dst_hbm, ew_hbm, out_hbm, idx_v, val_v, zb_v, deg_sh):
        c = lax.axis_index("c")
        s = lax.axis_index("s")
        zvec = jnp.zeros((16,), jnp.float32)

        def zfill(r, carry):
            zb_v[pl.ds(16 * r, 16)] = zvec
            return carry
        lax.fori_loop(0, zr // 16, zfill, 0)

        @pl.when(s < NS - 1)
        def _():
            pltpu.sync_copy(zb_v, deg_sh.at[pl.ds(s * zr, zr)])

        @pl.when(s == NS - 1)
        def _():
            rem = n - (NS - 1) * zr
            pltpu.sync_copy(zb_v.at[pl.ds(0, rem)],
                            deg_sh.at[pl.ds((NS - 1) * zr, rem)])
        plsc.subcore_barrier()

        base = (c * NS + s) * eb

        def blk(i, carry):
            off = base + i * LB
            pltpu.sync_copy(dst_hbm.at[pl.ds(off, LB)], idx_v)
            pltpu.sync_copy(ew_hbm.at[pl.ds(off, LB)], val_v)
            pltpu.sync_copy(val_v, deg_sh.at[idx_v], add=True)
            return carry
        lax.fori_loop(0, nb, blk, 0)
        plsc.subcore_barrier()

        @pl.when(s < NS - 1)
        def _():
            pltpu.sync_copy(deg_sh.at[pl.ds(s * zr, zr)], zb_v)
            pltpu.sync_copy(zb_v, out_hbm.at[pl.ds(c * n + s * zr, zr)])

        @pl.when(s == NS - 1)
        def _():
            rem = n - (NS - 1) * zr
            pltpu.sync_copy(deg_sh.at[pl.ds((NS - 1) * zr, rem)],
                            zb_v.at[pl.ds(0, rem)])
            pltpu.sync_copy(zb_v.at[pl.ds(0, rem)],
                            out_hbm.at[pl.ds(c * n + (NS - 1) * zr, rem)])

    return deg_kernel


def _make_agg_kernel(n, feat, eps, feature_split):
    nb = eps // LB
    zr = 640
    rem = n - (NS - 1) * zr
    nv = feat // 16
    mesh = plsc.VectorSubcoreMesh(core_axis_name="c", subcore_axis_name="s")

    @functools.partial(
        pl.kernel,
        out_type=jax.ShapeDtypeStruct((NC * n, feat), jnp.float32),
        mesh=mesh,
        scratch_types=[
            pltpu.VMEM((LB,), jnp.int32),
            pltpu.VMEM((LB,), jnp.int32),
            pltpu.VMEM((LB,), jnp.float32),
            pltpu.VMEM((LB, feat), jnp.float32),
            pltpu.VMEM_SHARED((n, feat), jnp.float32),
            pltpu.SemaphoreType.DMA,
        ],
    )
    def agg_kernel(tab_hbm, src_hbm, dst_hbm, ew_hbm, out_hbm,
                   sidx_v, didx_v, ew_v, rows_v, agg_sh, sem):
        c = lax.axis_index("c")
        s = lax.axis_index("s")
        zvec = jnp.zeros((16,), jnp.float32)

        def zfill(r, carry):
            for j in range(nv):
                rows_v[r, pl.ds(16 * j, 16)] = zvec
            return carry
        lax.fori_loop(0, LB, zfill, 0)

        @pl.when(s < NS - 1)
        def _():
            for t in range(zr // LB):
                pltpu.sync_copy(rows_v, agg_sh.at[pl.ds(s * zr + t * LB, LB)])

        @pl.when(s == NS - 1)
        def _():
            for t in range(rem // LB):
                pltpu.sync_copy(
                    rows_v, agg_sh.at[pl.ds((NS - 1) * zr + t * LB, LB)])
        plsc.subcore_barrier()

        if feature_split:
            base = s * eps
        else:
            base = (c * NS + s) * eps

        def blk(i, carry):
            off = base + i * LB
            pltpu.sync_copy(src_hbm.at[pl.ds(off, LB)], sidx_v)
            if feature_split:
                shift = c * n
                for k in range(LB // 16):
                    sidx_v[pl.ds(16 * k, 16)] = sidx_v[pl.ds(16 * k, 16)] + shift
            pltpu.async_copy(tab_hbm.at[sidx_v], rows_v, sem).wait()
            pltpu.sync_copy(ew_hbm.at[pl.ds(off, LB)], ew_v)
            pltpu.sync_copy(dst_hbm.at[pl.ds(off, LB)], didx_v)

            dnums = lax.GatherDimensionNumbers(
                offset_dims=(), collapsed_slice_dims=(0,), start_index_map=(0,))

            def scale(g, carry):
                ew16 = ew_v[pl.ds(16 * g, 16)]
                for lane in range(16):
                    nrm = lax.gather(
                        ew16, jnp.full((16, 1), lane, jnp.int32), dnums,
                        slice_sizes=(1,),
                        mode=lax.GatherScatterMode.PROMISE_IN_BOUNDS)
                    eidx = 16 * g + lane
                    for j in range(nv):
                        rows_v[eidx, pl.ds(16 * j, 16)] = (
                            rows_v[eidx, pl.ds(16 * j, 16)] * nrm)
                return carry
            lax.fori_loop(0, LB // 16, scale, 0)
            pltpu.sync_copy(rows_v, agg_sh.at[didx_v], add=True)
            return carry
        lax.fori_loop(0, nb, blk, 0)
        plsc.subcore_barrier()

        @pl.when(s < NS - 1)
        def _():
            for t in range(zr // LB):
                off = s * zr + t * LB
                pltpu.sync_copy(agg_sh.at[pl.ds(off, LB)], rows_v)
                pltpu.sync_copy(rows_v, out_hbm.at[pl.ds(c * n + off, LB)])

        @pl.when(s == NS - 1)
        def _():
            for t in range(rem // LB):
                off = (NS - 1) * zr + t * LB
                pltpu.sync_copy(agg_sh.at[pl.ds(off, LB)], rows_v)
                pltpu.sync_copy(rows_v, out_hbm.at[pl.ds(c * n + off, LB)])

    return agg_kernel


def _dinv_from(dp_blk):
    deg = 1.0 + dp_blk[0, 0, :] + dp_blk[0, 1, :]
    return jnp.where(deg > 0, lax.rsqrt(deg), 0.0)


def _tc1(x, W1, degp3):
    n, d = x.shape
    h = W1.shape[1]
    br = 1000

    def body(x_ref, w_ref, dp_ref, out_ref):
        dinv = _dinv_from(dp_ref)
        g = jnp.dot(x_ref[...], w_ref[...],
                    preferred_element_type=jnp.float32) * dinv[:, None]
        out_ref[0] = g[:, :h // 2]
        out_ref[1] = g[:, h // 2:]

    return pl.pallas_call(
        body,
        grid=(n // br,),
        in_specs=[
            pl.BlockSpec((br, d), lambda i: (i, 0)),
            pl.BlockSpec((d, h), lambda i: (0, 0)),
            pl.BlockSpec((1, 2, br), lambda i: (i, 0, 0)),
        ],
        out_specs=pl.BlockSpec((2, br, h // 2), lambda i: (0, i, 0)),
        out_shape=jax.ShapeDtypeStruct((2, n, h // 2), jnp.float32),
    )(x, W1, degp3)


def _tc2(agg1, g1, degp3, b1, W2):
    _, n, hf = agg1.shape
    h = 2 * hf
    emb = W2.shape[1]
    br = 1000

    def body(a_ref, g_ref, dp_ref, b_ref, w_ref, out_ref):
        dinv = _dinv_from(dp_ref)
        za = (a_ref[0] + g_ref[0]) * dinv[:, None] + b_ref[0, :hf]
        zb = (a_ref[1] + g_ref[1]) * dinv[:, None] + b_ref[0, hf:]
        z = jax.nn.relu(jnp.concatenate([za, zb], axis=1))
        h2 = jnp.dot(z, w_ref[...], preferred_element_type=jnp.float32)
        out_ref[...] = h2 * dinv[:, None]

    return pl.pallas_call(
        body,
        grid=(n // br,),
        in_specs=[
            pl.BlockSpec((2, br, hf), lambda i: (0, i, 0)),
            pl.BlockSpec((2, br, hf), lambda i: (0, i, 0)),
            pl.BlockSpec((1, 2, br), lambda i: (i, 0, 0)),
            pl.BlockSpec((1, h), lambda i: (0, 0)),
            pl.BlockSpec((h, emb), lambda i: (0, 0)),
        ],
        out_specs=pl.BlockSpec((br, emb), lambda i: (i, 0)),
        out_shape=jax.ShapeDtypeStruct((n, emb), jnp.float32),
    )(agg1, g1, degp3, b1, W2)


def _tc3(agg2, g2, degp3, b2):
    _, n, emb = agg2.shape
    br = 1000

    def body(a_ref, g_ref, dp_ref, b_ref, out_ref):
        dinv = _dinv_from(dp_ref)
        out_ref[...] = ((a_ref[0] + a_ref[1] + g_ref[...]) * dinv[:, None]
                        + b_ref[0])

    return pl.pallas_call(
        body,
        grid=(n // br,),
        in_specs=[
            pl.BlockSpec((2, br, emb), lambda i: (0, i, 0)),
            pl.BlockSpec((br, emb), lambda i: (i, 0)),
            pl.BlockSpec((1, 2, br), lambda i: (i, 0, 0)),
            pl.BlockSpec((1, emb), lambda i: (0, 0)),
        ],
        out_specs=pl.BlockSpec((br, emb), lambda i: (i, 0)),
        out_shape=jax.ShapeDtypeStruct((n, emb), jnp.float32),
    )(agg2, g2, degp3, b2)


def kernel(x, edge_index, edge_weight, W1, b1, W2, b2):
    n, d = x.shape
    e = edge_index.shape[1]
    h = W1.shape[1]
    emb = W2.shape[1]
    src = edge_index[0]
    dst = edge_index[1]

    degp = _make_deg_kernel(n, e)(dst, edge_weight)
    degp3 = degp.reshape(NC, n // 1000, 1000).transpose(1, 0, 2)

    g1 = _tc1(x, W1, degp3)
    g1cat = g1.reshape(NC * n, h // 2)
    agg1 = _make_agg_kernel(n, h // 2, e // NS, True)(
        g1cat, src, dst, edge_weight).reshape(NC, n, h // 2)

    g2 = _tc2(agg1, g1, degp3, b1.reshape(1, h), W2)
    agg2 = _make_agg_kernel(n, emb, e // (NC * NS), False)(
        g2, src, dst, edge_weight).reshape(NC, n, emb)

    return _tc3(agg2, g2, degp3, b2.reshape(1, emb))


# speedup vs baseline: 21.0675x; 2.6483x over previous
import functools

import jax
import jax.numpy as jnp
from jax import lax
from jax.experimental import pallas as pl
from jax.experimental.pallas import tpu as pltpu
from jax.experimental.pallas import tpu_sc as plsc

NC = 2
NS = 16
LB = 80
ZR = 640

_DNUMS = lax.GatherDimensionNumbers(
    offset_dims=(), collapsed_slice_dims=(0,), start_index_map=(0,))


def _make_deg_kernel(n, e):
    eps = e // (NC * NS)
    nb = eps // LB
    rem = n - (NS - 1) * ZR
    wdepth = 8
    mesh = plsc.VectorSubcoreMesh(core_axis_name="c", subcore_axis_name="s")

    @functools.partial(
        pl.kernel,
        out_type=jax.ShapeDtypeStruct((NC * n,), jnp.float32),
        mesh=mesh,
        scratch_types=[
            pltpu.VMEM((nb, LB), jnp.int32),
            pltpu.VMEM((eps,), jnp.float32),
            pltpu.VMEM((ZR,), jnp.float32),
            pltpu.VMEM_SHARED((n,), jnp.float32),
            pltpu.SemaphoreType.DMA,
        ],
    )
    def deg_kernel(dst2_hbm, ew2_hbm, out_hbm, didx_v, ew_v, zb_v, deg_sh, sem):
        c = lax.axis_index("c")
        s = lax.axis_index("s")
        w = c * NS + s
        pltpu.sync_copy(dst2_hbm.at[w], didx_v)
        pltpu.sync_copy(ew2_hbm.at[w], ew_v)

        zvec = jnp.zeros((16,), jnp.float32)

        def zfill(r, carry):
            zb_v[pl.ds(16 * r, 16)] = zvec
            return carry
        lax.fori_loop(0, ZR // 16, zfill, 0)

        @pl.when(s < NS - 1)
        def _():
            pltpu.sync_copy(zb_v, deg_sh.at[pl.ds(s * ZR, ZR)])

        @pl.when(s == NS - 1)
        def _():
            pltpu.sync_copy(zb_v.at[pl.ds(0, rem)],
                            deg_sh.at[pl.ds((NS - 1) * ZR, rem)])
        plsc.subcore_barrier()

        def sstart(k):
            pltpu.async_copy(ew_v.at[pl.ds(k * LB, LB)],
                             deg_sh.at[didx_v.at[k]], sem, add=True)

        def swait():
            pltpu.make_async_copy(ew_v.at[pl.ds(0, LB)],
                                  deg_sh.at[didx_v.at[0]], sem).wait()

        for k in range(wdepth):
            sstart(k)

        def body(k, carry):
            swait()
            sstart(k)
            return carry
        lax.fori_loop(wdepth, nb, body, 0)
        for _k in range(wdepth):
            swait()
        plsc.subcore_barrier()

        @pl.when(s < NS - 1)
        def _():
            pltpu.sync_copy(deg_sh.at[pl.ds(s * ZR, ZR)], zb_v)
            pltpu.sync_copy(zb_v, out_hbm.at[pl.ds(c * n + s * ZR, ZR)])

        @pl.when(s == NS - 1)
        def _():
            pltpu.sync_copy(deg_sh.at[pl.ds((NS - 1) * ZR, rem)],
                            zb_v.at[pl.ds(0, rem)])
            pltpu.sync_copy(zb_v.at[pl.ds(0, rem)],
                            out_hbm.at[pl.ds(c * n + (NS - 1) * ZR, rem)])

    return deg_kernel


def _make_agg_kernel(n, feat, eps, feature_split):
    nb = eps // LB
    rem = n - (NS - 1) * ZR
    nv = feat // 16
    mesh = plsc.VectorSubcoreMesh(core_axis_name="c", subcore_axis_name="s")

    @functools.partial(
        pl.kernel,
        out_type=jax.ShapeDtypeStruct((NC * n, feat), jnp.float32),
        mesh=mesh,
        scratch_types=[
            pltpu.VMEM((LB, feat), jnp.float32),
            pltpu.VMEM((LB, feat), jnp.float32),
            pltpu.VMEM((LB, feat), jnp.float32),
            pltpu.VMEM((LB,), jnp.int32),
            pltpu.VMEM((LB,), jnp.int32),
            pltpu.VMEM((LB,), jnp.int32),
            pltpu.VMEM((LB,), jnp.int32),
            pltpu.VMEM((LB,), jnp.float32),
            pltpu.VMEM((LB,), jnp.float32),
            pltpu.VMEM_SHARED((n, feat), jnp.float32),
            pltpu.SemaphoreType.DMA,
            pltpu.SemaphoreType.DMA,
            pltpu.SemaphoreType.DMA,
            pltpu.SemaphoreType.DMA,
            pltpu.SemaphoreType.DMA,
            pltpu.SemaphoreType.DMA,
            pltpu.SemaphoreType.DMA,
            pltpu.SemaphoreType.DMA,
        ],
    )
    def agg_kernel(tab_hbm, src_r, dst_r, ew_r, out_hbm,
                   rows_a, rows_b, zb_v, sidx_a, sidx_b, didx_a, didx_b,
                   ew_a, ew_b, agg_sh,
                   gs_a, gs_b, si_a, si_b, dew_a, dew_b, ss_a, ss_b):
        c = lax.axis_index("c")
        s = lax.axis_index("s")
        if feature_split:
            w = s
            shift = c * n
        else:
            w = c * NS + s
            shift = None
        rowbase = w * nb

        A = (rows_a, sidx_a, didx_a, ew_a, gs_a, si_a, dew_a, ss_a)
        B = (rows_b, sidx_b, didx_b, ew_b, gs_b, si_b, dew_b, ss_b)

        def sistart(X, k):
            pltpu.async_copy(src_r.at[rowbase + k], X[1], X[5])

        def siwait(X):
            pltpu.make_async_copy(src_r.at[rowbase], X[1], X[5]).wait()

        def doshift(X):
            if feature_split:
                for g in range(LB // 16):
                    X[1][pl.ds(16 * g, 16)] = X[1][pl.ds(16 * g, 16)] + shift

        def dewstart(X, k):
            pltpu.async_copy(dst_r.at[rowbase + k], X[2], X[6])
            pltpu.async_copy(ew_r.at[rowbase + k], X[3], X[6])

        def dewwait(X):
            pltpu.make_async_copy(dst_r.at[rowbase], X[2], X[6]).wait()
            pltpu.make_async_copy(ew_r.at[rowbase], X[3], X[6]).wait()

        def gstart(X):
            pltpu.async_copy(tab_hbm.at[X[1]], X[0], X[4])

        def gwait(X):
            pltpu.make_async_copy(tab_hbm.at[X[1]], X[0], X[4]).wait()

        def sstart(X):
            pltpu.async_copy(X[0], agg_sh.at[X[2]], X[7], add=True)

        def swait(X):
            pltpu.make_async_copy(X[0], agg_sh.at[X[2]], X[7]).wait()

        def scale(X):
            rows, ew_v = X[0], X[3]

            def grp(g, carry):
                ew16 = ew_v[pl.ds(16 * g, 16)]
                for lane in range(16):
                    nrm = lax.gather(
                        ew16, jnp.full((16, 1), lane, jnp.int32), _DNUMS,
                        slice_sizes=(1,),
                        mode=lax.GatherScatterMode.PROMISE_IN_BOUNDS)
                    eidx = 16 * g + lane
                    for j in range(nv):
                        rows[eidx, pl.ds(16 * j, 16)] = (
                            rows[eidx, pl.ds(16 * j, 16)] * nrm)
                return carry
            lax.fori_loop(0, LB // 16, grp, 0)

        def phase(k, X, Y, first=False):
            static = isinstance(k, int)
            gwait(X)

            def ahead2():
                sistart(X, k + 2)

            if static:
                if k + 2 < nb:
                    ahead2()
            else:
                pl.when(k + 2 < nb)(ahead2)
            if not first:
                swait(Y)

            def ahead1():
                siwait(Y)
                doshift(Y)
                gstart(Y)
                dewstart(Y, k + 1)

            if static:
                if k + 1 < nb:
                    ahead1()
            else:
                pl.when(k + 1 < nb)(ahead1)
            dewwait(X)
            scale(X)
            sstart(X)

        sistart(A, 0)
        dewstart(A, 0)
        siwait(A)
        doshift(A)
        gstart(A)
        sistart(B, 1)

        zvec = jnp.zeros((16,), jnp.float32)

        def zfill(r, carry):
            for j in range(nv):
                zb_v[r, pl.ds(16 * j, 16)] = zvec
            return carry
        lax.fori_loop(0, LB, zfill, 0)

        @pl.when(s < NS - 1)
        def _():
            for t in range(ZR // LB):
                pltpu.sync_copy(zb_v, agg_sh.at[pl.ds(s * ZR + t * LB, LB)])

        @pl.when(s == NS - 1)
        def _():
            for t in range(rem // LB):
                pltpu.sync_copy(
                    zb_v, agg_sh.at[pl.ds((NS - 1) * ZR + t * LB, LB)])
        plsc.subcore_barrier()

        phase(0, A, B, first=True)
        phase(1, B, A)
        npairs = (nb - 2) // 2

        def body(t, carry):
            k = 2 + 2 * t
            phase(k, A, B)
            phase(k + 1, B, A)
            return carry
        lax.fori_loop(0, npairs, body, 0)
        if (nb - 2) % 2 == 1:
            phase(nb - 1, A, B)
            swait(A)
        else:
            swait(B)
        plsc.subcore_barrier()

        @pl.when(s < NS - 1)
        def _():
            for t in range(ZR // LB):
                off = s * ZR + t * LB
                pltpu.sync_copy(agg_sh.at[pl.ds(off, LB)], zb_v)
                pltpu.sync_copy(zb_v, out_hbm.at[pl.ds(c * n + off, LB)])

        @pl.when(s == NS - 1)
        def _():
            for t in range(rem // LB):
                off = (NS - 1) * ZR + t * LB
                pltpu.sync_copy(agg_sh.at[pl.ds(off, LB)], zb_v)
                pltpu.sync_copy(zb_v, out_hbm.at[pl.ds(c * n + off, LB)])

    return agg_kernel


def _dinv_from(dp_blk):
    deg = 1.0 + dp_blk[0, 0, :] + dp_blk[0, 1, :]
    return jnp.where(deg > 0, lax.rsqrt(deg), 0.0)


def _tc1(x, W1, degp3):
    n, d = x.shape
    h = W1.shape[1]
    br = 1000

    def body(x_ref, w_ref, dp_ref, out_ref):
        dinv = _dinv_from(dp_ref)
        g = jnp.dot(x_ref[...], w_ref[...],
                    preferred_element_type=jnp.float32) * dinv[:, None]
        out_ref[0] = g[:, :h // 2]
        out_ref[1] = g[:, h // 2:]

    return pl.pallas_call(
        body,
        grid=(n // br,),
        in_specs=[
            pl.BlockSpec((br, d), lambda i: (i, 0)),
            pl.BlockSpec((d, h), lambda i: (0, 0)),
            pl.BlockSpec((1, 2, br), lambda i: (i, 0, 0)),
        ],
        out_specs=pl.BlockSpec((2, br, h // 2), lambda i: (0, i, 0)),
        out_shape=jax.ShapeDtypeStruct((2, n, h // 2), jnp.float32),
    )(x, W1, degp3)


def _tc2(agg1, g1, degp3, b1, W2):
    _, n, hf = agg1.shape
    h = 2 * hf
    emb = W2.shape[1]
    br = 1000

    def body(a_ref, g_ref, dp_ref, b_ref, w_ref, out_ref):
        dinv = _dinv_from(dp_ref)
        za = (a_ref[0] + g_ref[0]) * dinv[:, None] + b_ref[0, :hf]
        zb = (a_ref[1] + g_ref[1]) * dinv[:, None] + b_ref[0, hf:]
        z = jax.nn.relu(jnp.concatenate([za, zb], axis=1))
        h2 = jnp.dot(z, w_ref[...], preferred_element_type=jnp.float32)
        out_ref[...] = h2 * dinv[:, None]

    return pl.pallas_call(
        body,
        grid=(n // br,),
        in_specs=[
            pl.BlockSpec((2, br, hf), lambda i: (0, i, 0)),
            pl.BlockSpec((2, br, hf), lambda i: (0, i, 0)),
            pl.BlockSpec((1, 2, br), lambda i: (i, 0, 0)),
            pl.BlockSpec((1, h), lambda i: (0, 0)),
            pl.BlockSpec((h, emb), lambda i: (0, 0)),
        ],
        out_specs=pl.BlockSpec((br, emb), lambda i: (i, 0)),
        out_shape=jax.ShapeDtypeStruct((n, emb), jnp.float32),
    )(agg1, g1, degp3, b1, W2)


def _tc3(agg2, g2, degp3, b2):
    _, n, emb = agg2.shape
    br = 1000

    def body(a_ref, g_ref, dp_ref, b_ref, out_ref):
        dinv = _dinv_from(dp_ref)
        out_ref[...] = ((a_ref[0] + a_ref[1] + g_ref[...]) * dinv[:, None]
                        + b_ref[0])

    return pl.pallas_call(
        body,
        grid=(n // br,),
        in_specs=[
            pl.BlockSpec((2, br, emb), lambda i: (0, i, 0)),
            pl.BlockSpec((br, emb), lambda i: (i, 0)),
            pl.BlockSpec((1, 2, br), lambda i: (i, 0, 0)),
            pl.BlockSpec((1, emb), lambda i: (0, 0)),
        ],
        out_specs=pl.BlockSpec((br, emb), lambda i: (i, 0)),
        out_shape=jax.ShapeDtypeStruct((n, emb), jnp.float32),
    )(agg2, g2, degp3, b2)


def kernel(x, edge_index, edge_weight, W1, b1, W2, b2):
    n, d = x.shape
    e = edge_index.shape[1]
    h = W1.shape[1]
    emb = W2.shape[1]
    src = edge_index[0]
    dst = edge_index[1]

    eps1 = e // NS
    eps2 = e // (NC * NS)

    src_r = src.reshape(e // LB, LB)
    dst_r = dst.reshape(e // LB, LB)
    ew_r = edge_weight.reshape(e // LB, LB)
    dst2 = dst.reshape(NC * NS, eps2 // LB, LB)
    ew2 = edge_weight.reshape(NC * NS, eps2)

    degp = _make_deg_kernel(n, e)(dst2, ew2)
    degp3 = degp.reshape(NC, n // 1000, 1000).transpose(1, 0, 2)

    g1 = _tc1(x, W1, degp3)
    g1cat = g1.reshape(NC * n, h // 2)
    agg1 = _make_agg_kernel(n, h // 2, eps1, True)(
        g1cat, src_r, dst_r, ew_r).reshape(NC, n, h // 2)

    g2 = _tc2(agg1, g1, degp3, b1.reshape(1, h), W2)
    agg2 = _make_agg_kernel(n, emb, eps2, False)(
        g2, src_r, dst_r, ew_r).reshape(NC, n, emb)

    return _tc3(agg2, g2, degp3, b2.reshape(1, emb))


# speedup vs baseline: 28.8852x; 1.3711x over previous
import functools

import jax
import jax.numpy as jnp
from jax import lax
from jax.experimental import pallas as pl
from jax.experimental.pallas import tpu as pltpu
from jax.experimental.pallas import tpu_sc as plsc

NC = 2
NS = 16
LB = 80
ZR = 640

_DNUMS = lax.GatherDimensionNumbers(
    offset_dims=(), collapsed_slice_dims=(0,), start_index_map=(0,))


def _make_deg_kernel(n, e):
    eps = e // (NC * NS)
    nb = eps // LB
    rem = n - (NS - 1) * ZR
    wdepth = 8
    mesh = plsc.VectorSubcoreMesh(core_axis_name="c", subcore_axis_name="s")

    @functools.partial(
        pl.kernel,
        out_type=jax.ShapeDtypeStruct((NC * n,), jnp.float32),
        mesh=mesh,
        scratch_types=[
            pltpu.VMEM((nb, LB), jnp.int32),
            pltpu.VMEM((eps,), jnp.float32),
            pltpu.VMEM((ZR,), jnp.float32),
            pltpu.VMEM_SHARED((n,), jnp.float32),
            pltpu.SemaphoreType.DMA,
        ],
    )
    def deg_kernel(dst2_hbm, ew2_hbm, out_hbm, didx_v, ew_v, zb_v, deg_sh, sem):
        c = lax.axis_index("c")
        s = lax.axis_index("s")
        w = c * NS + s
        pltpu.sync_copy(dst2_hbm.at[w], didx_v)
        pltpu.sync_copy(ew2_hbm.at[w], ew_v)

        zvec = jnp.zeros((16,), jnp.float32)

        def zfill(r, carry):
            zb_v[pl.ds(16 * r, 16)] = zvec
            return carry
        lax.fori_loop(0, ZR // 16, zfill, 0)

        @pl.when(s < NS - 1)
        def _():
            pltpu.sync_copy(zb_v, deg_sh.at[pl.ds(s * ZR, ZR)])

        @pl.when(s == NS - 1)
        def _():
            pltpu.sync_copy(zb_v.at[pl.ds(0, rem)],
                            deg_sh.at[pl.ds((NS - 1) * ZR, rem)])
        plsc.subcore_barrier()

        def sstart(k):
            pltpu.async_copy(ew_v.at[pl.ds(k * LB, LB)],
                             deg_sh.at[didx_v.at[k]], sem, add=True)

        def swait():
            pltpu.make_async_copy(ew_v.at[pl.ds(0, LB)],
                                  deg_sh.at[didx_v.at[0]], sem).wait()

        for k in range(wdepth):
            sstart(k)

        def body(k, carry):
            swait()
            sstart(k)
            return carry
        lax.fori_loop(wdepth, nb, body, 0)
        for _k in range(wdepth):
            swait()
        plsc.subcore_barrier()

        @pl.when(s < NS - 1)
        def _():
            pltpu.sync_copy(deg_sh.at[pl.ds(s * ZR, ZR)], zb_v)
            pltpu.sync_copy(zb_v, out_hbm.at[pl.ds(c * n + s * ZR, ZR)])

        @pl.when(s == NS - 1)
        def _():
            pltpu.sync_copy(deg_sh.at[pl.ds((NS - 1) * ZR, rem)],
                            zb_v.at[pl.ds(0, rem)])
            pltpu.sync_copy(zb_v.at[pl.ds(0, rem)],
                            out_hbm.at[pl.ds(c * n + (NS - 1) * ZR, rem)])

    return deg_kernel


def _make_agg_kernel(n, feat, eps):
    nb = eps // LB
    rem = n - (NS - 1) * ZR
    nv = feat // 16
    mesh = plsc.VectorSubcoreMesh(core_axis_name="c", subcore_axis_name="s")

    @functools.partial(
        pl.kernel,
        out_type=jax.ShapeDtypeStruct((NC * n, feat), jnp.float32),
        mesh=mesh,
        scratch_types=[
            pltpu.VMEM((LB, feat), jnp.float32),
            pltpu.VMEM((LB, feat), jnp.float32),
            pltpu.VMEM((LB, feat), jnp.float32),
            pltpu.VMEM((LB,), jnp.int32),
            pltpu.VMEM((LB,), jnp.int32),
            pltpu.VMEM((LB,), jnp.int32),
            pltpu.VMEM((LB,), jnp.int32),
            pltpu.VMEM((LB,), jnp.float32),
            pltpu.VMEM((LB,), jnp.float32),
            pltpu.VMEM_SHARED((n, feat), jnp.float32),
            pltpu.SemaphoreType.DMA,
            pltpu.SemaphoreType.DMA,
            pltpu.SemaphoreType.DMA,
            pltpu.SemaphoreType.DMA,
            pltpu.SemaphoreType.DMA,
            pltpu.SemaphoreType.DMA,
            pltpu.SemaphoreType.DMA,
            pltpu.SemaphoreType.DMA,
        ],
    )
    def agg_kernel(tab_hbm, src_r, dst_r, ew_r, out_hbm,
                   rows_a, rows_b, zb_v, sidx_a, sidx_b, didx_a, didx_b,
                   ew_a, ew_b, agg_sh,
                   gs_a, gs_b, si_a, si_b, dew_a, dew_b, ss_a, ss_b):
        c = lax.axis_index("c")
        s = lax.axis_index("s")
        w = c * NS + s
        rowbase = w * nb

        A = (rows_a, sidx_a, didx_a, ew_a, gs_a, si_a, dew_a, ss_a)
        B = (rows_b, sidx_b, didx_b, ew_b, gs_b, si_b, dew_b, ss_b)

        def sistart(X, k):
            pltpu.async_copy(src_r.at[rowbase + k], X[1], X[5])

        def siwait(X):
            pltpu.make_async_copy(src_r.at[rowbase], X[1], X[5]).wait()

        def dewstart(X, k):
            pltpu.async_copy(dst_r.at[rowbase + k], X[2], X[6])
            pltpu.async_copy(ew_r.at[rowbase + k], X[3], X[6])

        def dewwait(X):
            pltpu.make_async_copy(dst_r.at[rowbase], X[2], X[6]).wait()
            pltpu.make_async_copy(ew_r.at[rowbase], X[3], X[6]).wait()

        def gstart(X):
            pltpu.async_copy(tab_hbm.at[X[1]], X[0], X[4])

        def gwait(X):
            pltpu.make_async_copy(tab_hbm.at[X[1]], X[0], X[4]).wait()

        def sstart(X):
            pltpu.async_copy(X[0], agg_sh.at[X[2]], X[7], add=True)

        def swait(X):
            pltpu.make_async_copy(X[0], agg_sh.at[X[2]], X[7]).wait()

        def scale(X):
            rows, ew_v = X[0], X[3]

            def grp(g, carry):
                ew16 = ew_v[pl.ds(16 * g, 16)]
                for lane in range(16):
                    nrm = lax.gather(
                        ew16, jnp.full((16, 1), lane, jnp.int32), _DNUMS,
                        slice_sizes=(1,),
                        mode=lax.GatherScatterMode.PROMISE_IN_BOUNDS)
                    eidx = 16 * g + lane
                    for j in range(nv):
                        rows[eidx, pl.ds(16 * j, 16)] = (
                            rows[eidx, pl.ds(16 * j, 16)] * nrm)
                return carry
            lax.fori_loop(0, LB // 16, grp, 0)

        def phase(k, X, Y, first=False):
            static = isinstance(k, int)
            gwait(X)

            def ahead2():
                sistart(X, k + 2)

            if static:
                if k + 2 < nb:
                    ahead2()
            else:
                pl.when(k + 2 < nb)(ahead2)
            if not first:
                swait(Y)

            def ahead1():
                siwait(Y)
                gstart(Y)
                dewstart(Y, k + 1)

            if static:
                if k + 1 < nb:
                    ahead1()
            else:
                pl.when(k + 1 < nb)(ahead1)
            dewwait(X)
            scale(X)
            sstart(X)

        sistart(A, 0)
        dewstart(A, 0)
        siwait(A)
        gstart(A)
        sistart(B, 1)

        zvec = jnp.zeros((16,), jnp.float32)

        def zfill(r, carry):
            for j in range(nv):
                zb_v[r, pl.ds(16 * j, 16)] = zvec
            return carry
        lax.fori_loop(0, LB, zfill, 0)

        @pl.when(s < NS - 1)
        def _():
            for t in range(ZR // LB):
                pltpu.sync_copy(zb_v, agg_sh.at[pl.ds(s * ZR + t * LB, LB)])

        @pl.when(s == NS - 1)
        def _():
            for t in range(rem // LB):
                pltpu.sync_copy(
                    zb_v, agg_sh.at[pl.ds((NS - 1) * ZR + t * LB, LB)])
        plsc.subcore_barrier()

        phase(0, A, B, first=True)
        phase(1, B, A)
        npairs = (nb - 2) // 2

        def body(t, carry):
            k = 2 + 2 * t
            phase(k, A, B)
            phase(k + 1, B, A)
            return carry
        lax.fori_loop(0, npairs, body, 0)
        if (nb - 2) % 2 == 1:
            phase(nb - 1, A, B)
            swait(A)
        else:
            swait(B)
        plsc.subcore_barrier()

        @pl.when(s < NS - 1)
        def _():
            for t in range(ZR // LB):
                off = s * ZR + t * LB
                pltpu.sync_copy(agg_sh.at[pl.ds(off, LB)], zb_v)
                pltpu.sync_copy(zb_v, out_hbm.at[pl.ds(c * n + off, LB)])

        @pl.when(s == NS - 1)
        def _():
            for t in range(rem // LB):
                off = (NS - 1) * ZR + t * LB
                pltpu.sync_copy(agg_sh.at[pl.ds(off, LB)], zb_v)
                pltpu.sync_copy(zb_v, out_hbm.at[pl.ds(c * n + off, LB)])

    return agg_kernel


def _dinv_from(dp_blk):
    deg = 1.0 + dp_blk[0, 0, :] + dp_blk[0, 1, :]
    return jnp.where(deg > 0, lax.rsqrt(deg), 0.0)


def _tc1(x, degp3):
    n, d = x.shape
    br = 1000

    def body(x_ref, dp_ref, out_ref):
        dinv = _dinv_from(dp_ref)
        out_ref[...] = x_ref[...] * dinv[:, None]

    return pl.pallas_call(
        body,
        grid=(n // br,),
        in_specs=[
            pl.BlockSpec((br, d), lambda i: (i, 0)),
            pl.BlockSpec((1, 2, br), lambda i: (i, 0, 0)),
        ],
        out_specs=pl.BlockSpec((br, d), lambda i: (i, 0)),
        out_shape=jax.ShapeDtypeStruct((n, d), jnp.float32),
    )(x, degp3)


def _tc2(agg0, g0, degp3, W1, b1, W2):
    _, n, d = agg0.shape
    h = W1.shape[1]
    emb = W2.shape[1]
    br = 1000

    def body(a_ref, g_ref, dp_ref, w1_ref, b_ref, w2_ref, out_ref):
        dinv = _dinv_from(dp_ref)
        ax = (a_ref[0] + a_ref[1] + g_ref[...]) * dinv[:, None]
        z = jax.nn.relu(
            jnp.dot(ax, w1_ref[...], preferred_element_type=jnp.float32)
            + b_ref[0])
        h2 = jnp.dot(z, w2_ref[...], preferred_element_type=jnp.float32)
        out_ref[...] = h2 * dinv[:, None]

    return pl.pallas_call(
        body,
        grid=(n // br,),
        in_specs=[
            pl.BlockSpec((2, br, d), lambda i: (0, i, 0)),
            pl.BlockSpec((br, d), lambda i: (i, 0)),
            pl.BlockSpec((1, 2, br), lambda i: (i, 0, 0)),
            pl.BlockSpec((d, h), lambda i: (0, 0)),
            pl.BlockSpec((1, h), lambda i: (0, 0)),
            pl.BlockSpec((h, emb), lambda i: (0, 0)),
        ],
        out_specs=pl.BlockSpec((br, emb), lambda i: (i, 0)),
        out_shape=jax.ShapeDtypeStruct((n, emb), jnp.float32),
    )(agg0, g0, degp3, W1, b1, W2)


def _tc3(agg2, g2, degp3, b2):
    _, n, emb = agg2.shape
    br = 1000

    def body(a_ref, g_ref, dp_ref, b_ref, out_ref):
        dinv = _dinv_from(dp_ref)
        out_ref[...] = ((a_ref[0] + a_ref[1] + g_ref[...]) * dinv[:, None]
                        + b_ref[0])

    return pl.pallas_call(
        body,
        grid=(n // br,),
        in_specs=[
            pl.BlockSpec((2, br, emb), lambda i: (0, i, 0)),
            pl.BlockSpec((br, emb), lambda i: (i, 0)),
            pl.BlockSpec((1, 2, br), lambda i: (i, 0, 0)),
            pl.BlockSpec((1, emb), lambda i: (0, 0)),
        ],
        out_specs=pl.BlockSpec((br, emb), lambda i: (i, 0)),
        out_shape=jax.ShapeDtypeStruct((n, emb), jnp.float32),
    )(agg2, g2, degp3, b2)


def kernel(x, edge_index, edge_weight, W1, b1, W2, b2):
    n, d = x.shape
    e = edge_index.shape[1]
    h = W1.shape[1]
    emb = W2.shape[1]
    src = edge_index[0]
    dst = edge_index[1]

    eps = e // (NC * NS)

    src_r = src.reshape(e // LB, LB)
    dst_r = dst.reshape(e // LB, LB)
    ew_r = edge_weight.reshape(e // LB, LB)
    dst2 = dst.reshape(NC * NS, eps // LB, LB)
    ew2 = edge_weight.reshape(NC * NS, eps)

    degp = _make_deg_kernel(n, e)(dst2, ew2)
    degp3 = degp.reshape(NC, n // 1000, 1000).transpose(1, 0, 2)

    g0 = _tc1(x, degp3)
    agg0 = _make_agg_kernel(n, d, eps)(
        g0, src_r, dst_r, ew_r).reshape(NC, n, d)

    g2 = _tc2(agg0, g0, degp3, W1, b1.reshape(1, h), W2)
    agg2 = _make_agg_kernel(n, emb, eps)(
        g2, src_r, dst_r, ew_r).reshape(NC, n, emb)

    return _tc3(agg2, g2, degp3, b2.reshape(1, emb))


# speedup vs baseline: 35.2617x; 1.2208x over previous
import functools

import jax
import jax.numpy as jnp
from jax import lax
from jax.experimental import pallas as pl
from jax.experimental.pallas import tpu as pltpu
from jax.experimental.pallas import tpu_sc as plsc

NC = 2
NS = 16
LB = 80
ZR = 640

_DNUMS = lax.GatherDimensionNumbers(
    offset_dims=(), collapsed_slice_dims=(0,), start_index_map=(0,))


def _make_deg_kernel(n, e):
    eps = e // (NC * NS)
    nb = eps // LB
    rem = n - (NS - 1) * ZR
    wdepth = 8
    mesh = plsc.VectorSubcoreMesh(core_axis_name="c", subcore_axis_name="s")

    @functools.partial(
        pl.kernel,
        out_type=jax.ShapeDtypeStruct((NC * n,), jnp.float32),
        mesh=mesh,
        scratch_types=[
            pltpu.VMEM((nb, LB), jnp.int32),
            pltpu.VMEM((eps,), jnp.float32),
            pltpu.VMEM((ZR,), jnp.float32),
            pltpu.VMEM_SHARED((n,), jnp.float32),
            pltpu.SemaphoreType.DMA,
        ],
    )
    def deg_kernel(dst2_hbm, ew2_hbm, out_hbm, didx_v, ew_v, zb_v, deg_sh, sem):
        c = lax.axis_index("c")
        s = lax.axis_index("s")
        w = c * NS + s
        pltpu.sync_copy(dst2_hbm.at[w], didx_v)
        pltpu.sync_copy(ew2_hbm.at[w], ew_v)

        zvec = jnp.zeros((16,), jnp.float32)

        def zfill(r, carry):
            zb_v[pl.ds(16 * r, 16)] = zvec
            return carry
        lax.fori_loop(0, ZR // 16, zfill, 0)

        @pl.when(s < NS - 1)
        def _():
            pltpu.sync_copy(zb_v, deg_sh.at[pl.ds(s * ZR, ZR)])

        @pl.when(s == NS - 1)
        def _():
            pltpu.sync_copy(zb_v.at[pl.ds(0, rem)],
                            deg_sh.at[pl.ds((NS - 1) * ZR, rem)])
        plsc.subcore_barrier()

        def sstart(k):
            pltpu.async_copy(ew_v.at[pl.ds(k * LB, LB)],
                             deg_sh.at[didx_v.at[k]], sem, add=True)

        def swait():
            pltpu.make_async_copy(ew_v.at[pl.ds(0, LB)],
                                  deg_sh.at[didx_v.at[0]], sem).wait()

        for k in range(wdepth):
            sstart(k)

        def body(k, carry):
            swait()
            sstart(k)
            return carry
        lax.fori_loop(wdepth, nb, body, 0)
        for _k in range(wdepth):
            swait()
        plsc.subcore_barrier()

        @pl.when(s < NS - 1)
        def _():
            pltpu.sync_copy(deg_sh.at[pl.ds(s * ZR, ZR)], zb_v)
            pltpu.sync_copy(zb_v, out_hbm.at[pl.ds(c * n + s * ZR, ZR)])

        @pl.when(s == NS - 1)
        def _():
            pltpu.sync_copy(deg_sh.at[pl.ds((NS - 1) * ZR, rem)],
                            zb_v.at[pl.ds(0, rem)])
            pltpu.sync_copy(zb_v.at[pl.ds(0, rem)],
                            out_hbm.at[pl.ds(c * n + (NS - 1) * ZR, rem)])

    return deg_kernel


def _make_agg_kernel(n, feat, eps):
    nb = eps // LB
    assert nb >= 5 and (nb - 2) % 3 == 0, "phase schedule needs exact triples"
    rem = n - (NS - 1) * ZR
    nv = feat // 16
    mesh = plsc.VectorSubcoreMesh(core_axis_name="c", subcore_axis_name="s")

    @functools.partial(
        pl.kernel,
        out_type=jax.ShapeDtypeStruct((NC * n, feat), jnp.float32),
        mesh=mesh,
        scratch_types=[
            pltpu.VMEM((LB, feat), jnp.float32),
            pltpu.VMEM((LB, feat), jnp.float32),
            pltpu.VMEM((LB, feat), jnp.float32),
            pltpu.VMEM((LB,), jnp.int32),
            pltpu.VMEM((LB,), jnp.int32),
            pltpu.VMEM((LB,), jnp.int32),
            pltpu.VMEM((LB,), jnp.int32),
            pltpu.VMEM((LB,), jnp.int32),
            pltpu.VMEM((LB,), jnp.int32),
            pltpu.VMEM((LB,), jnp.float32),
            pltpu.VMEM((LB,), jnp.float32),
            pltpu.VMEM((LB,), jnp.float32),
            pltpu.VMEM_SHARED((n, feat), jnp.float32),
            pltpu.SemaphoreType.DMA,
            pltpu.SemaphoreType.DMA,
            pltpu.SemaphoreType.DMA,
            pltpu.SemaphoreType.DMA,
            pltpu.SemaphoreType.DMA,
            pltpu.SemaphoreType.DMA,
            pltpu.SemaphoreType.DMA,
            pltpu.SemaphoreType.DMA,
            pltpu.SemaphoreType.DMA,
            pltpu.SemaphoreType.DMA,
            pltpu.SemaphoreType.DMA,
            pltpu.SemaphoreType.DMA,
        ],
    )
    def agg_kernel(tab_hbm, src_r, dst_r, ew_r, out_hbm,
                   rows_a, rows_b, rows_c, sidx_a, sidx_b, sidx_c,
                   didx_a, didx_b, didx_c, ew_a, ew_b, ew_c, agg_sh,
                   gs_a, gs_b, gs_c, si_a, si_b, si_c,
                   dew_a, dew_b, dew_c, ss_a, ss_b, ss_c):
        c = lax.axis_index("c")
        s = lax.axis_index("s")
        w = c * NS + s
        rowbase = w * nb

        A = (rows_a, sidx_a, didx_a, ew_a, gs_a, si_a, dew_a, ss_a)
        B = (rows_b, sidx_b, didx_b, ew_b, gs_b, si_b, dew_b, ss_b)
        C = (rows_c, sidx_c, didx_c, ew_c, gs_c, si_c, dew_c, ss_c)
        zb_v = rows_c

        def sistart(X, k):
            pltpu.async_copy(src_r.at[rowbase + k], X[1], X[5])

        def siwait(X):
            pltpu.make_async_copy(src_r.at[rowbase], X[1], X[5]).wait()

        def dewstart(X, k):
            pltpu.async_copy(dst_r.at[rowbase + k], X[2], X[6])
            pltpu.async_copy(ew_r.at[rowbase + k], X[3], X[6])

        def dewwait(X):
            pltpu.make_async_copy(dst_r.at[rowbase], X[2], X[6]).wait()
            pltpu.make_async_copy(ew_r.at[rowbase], X[3], X[6]).wait()

        def gstart(X):
            pltpu.async_copy(tab_hbm.at[X[1]], X[0], X[4])

        def gwait(X):
            pltpu.make_async_copy(tab_hbm.at[X[1]], X[0], X[4]).wait()

        def sstart(X):
            pltpu.async_copy(X[0], agg_sh.at[X[2]], X[7], add=True)

        def swait(X):
            pltpu.make_async_copy(X[0], agg_sh.at[X[2]], X[7]).wait()

        def scale(X):
            rows, ew_v = X[0], X[3]

            def grp(g, carry):
                ew16 = ew_v[pl.ds(16 * g, 16)]
                for lane in range(16):
                    nrm = lax.gather(
                        ew16, jnp.full((16, 1), lane, jnp.int32), _DNUMS,
                        slice_sizes=(1,),
                        mode=lax.GatherScatterMode.PROMISE_IN_BOUNDS)
                    eidx = 16 * g + lane
                    for j in range(nv):
                        rows[eidx, pl.ds(16 * j, 16)] = (
                            rows[eidx, pl.ds(16 * j, 16)] * nrm)
                return carry
            lax.fori_loop(0, LB // 16, grp, 0)

        def phase(k, X, Z, first=False):
            static = isinstance(k, int)
            gwait(X)

            def ahead3():
                sistart(X, k + 3)

            if static:
                if k + 3 < nb:
                    ahead3()
            else:
                pl.when(k + 3 < nb)(ahead3)
            if not first:
                swait(Z)

            def ahead2():
                siwait(Z)
                gstart(Z)
                dewstart(Z, k + 2)

            if static:
                if k + 2 < nb:
                    ahead2()
            else:
                pl.when(k + 2 < nb)(ahead2)
            dewwait(X)
            scale(X)
            sstart(X)

        sistart(A, 0)
        dewstart(A, 0)
        sistart(B, 1)
        dewstart(B, 1)
        siwait(A)
        gstart(A)
        siwait(B)
        gstart(B)
        sistart(C, 2)

        zvec = jnp.zeros((16,), jnp.float32)

        def zfill(r, carry):
            for j in range(nv):
                zb_v[r, pl.ds(16 * j, 16)] = zvec
            return carry
        lax.fori_loop(0, LB, zfill, 0)

        @pl.when(s < NS - 1)
        def _():
            for t in range(ZR // LB):
                pltpu.sync_copy(zb_v, agg_sh.at[pl.ds(s * ZR + t * LB, LB)])

        @pl.when(s == NS - 1)
        def _():
            for t in range(rem // LB):
                pltpu.sync_copy(
                    zb_v, agg_sh.at[pl.ds((NS - 1) * ZR + t * LB, LB)])
        plsc.subcore_barrier()

        phase(0, A, C, first=True)
        phase(1, B, A)
        ntr = (nb - 2) // 3

        def body(t, carry):
            k = 2 + 3 * t
            phase(k, C, B)
            phase(k + 1, A, C)
            phase(k + 2, B, A)
            return carry
        lax.fori_loop(0, ntr, body, 0)
        swait((A, B, C)[(nb - 1) % 3])
        plsc.subcore_barrier()

        @pl.when(s < NS - 1)
        def _():
            for t in range(ZR // LB):
                off = s * ZR + t * LB
                pltpu.sync_copy(agg_sh.at[pl.ds(off, LB)], zb_v)
                pltpu.sync_copy(zb_v, out_hbm.at[pl.ds(c * n + off, LB)])

        @pl.when(s == NS - 1)
        def _():
            for t in range(rem // LB):
                off = (NS - 1) * ZR + t * LB
                pltpu.sync_copy(agg_sh.at[pl.ds(off, LB)], zb_v)
                pltpu.sync_copy(zb_v, out_hbm.at[pl.ds(c * n + off, LB)])

    return agg_kernel


def _dinv_from(dp_blk):
    deg = 1.0 + dp_blk[0, 0, :] + dp_blk[0, 1, :]
    return jnp.where(deg > 0, lax.rsqrt(deg), 0.0)


def _tc1(x, degp3):
    n, d = x.shape
    br = 1000

    def body(x_ref, dp_ref, out_ref):
        dinv = _dinv_from(dp_ref)
        out_ref[...] = x_ref[...] * dinv[:, None]

    return pl.pallas_call(
        body,
        grid=(n // br,),
        in_specs=[
            pl.BlockSpec((br, d), lambda i: (i, 0)),
            pl.BlockSpec((1, 2, br), lambda i: (i, 0, 0)),
        ],
        out_specs=pl.BlockSpec((br, d), lambda i: (i, 0)),
        out_shape=jax.ShapeDtypeStruct((n, d), jnp.float32),
    )(x, degp3)


def _tc2(agg0, g0, degp3, W1, b1, W2):
    _, n, d = agg0.shape
    h = W1.shape[1]
    emb = W2.shape[1]
    br = 1000

    def body(a_ref, g_ref, dp_ref, w1_ref, b_ref, w2_ref, out_ref):
        dinv = _dinv_from(dp_ref)
        ax = (a_ref[0] + a_ref[1] + g_ref[...]) * dinv[:, None]
        z = jax.nn.relu(
            jnp.dot(ax, w1_ref[...], preferred_element_type=jnp.float32)
            + b_ref[0])
        h2 = jnp.dot(z, w2_ref[...], preferred_element_type=jnp.float32)
        out_ref[...] = h2 * dinv[:, None]

    return pl.pallas_call(
        body,
        grid=(n // br,),
        in_specs=[
            pl.BlockSpec((2, br, d), lambda i: (0, i, 0)),
            pl.BlockSpec((br, d), lambda i: (i, 0)),
            pl.BlockSpec((1, 2, br), lambda i: (i, 0, 0)),
            pl.BlockSpec((d, h), lambda i: (0, 0)),
            pl.BlockSpec((1, h), lambda i: (0, 0)),
            pl.BlockSpec((h, emb), lambda i: (0, 0)),
        ],
        out_specs=pl.BlockSpec((br, emb), lambda i: (i, 0)),
        out_shape=jax.ShapeDtypeStruct((n, emb), jnp.float32),
    )(agg0, g0, degp3, W1, b1, W2)


def _tc3(agg2, g2, degp3, b2):
    _, n, emb = agg2.shape
    br = 1000

    def body(a_ref, g_ref, dp_ref, b_ref, out_ref):
        dinv = _dinv_from(dp_ref)
        out_ref[...] = ((a_ref[0] + a_ref[1] + g_ref[...]) * dinv[:, None]
                        + b_ref[0])

    return pl.pallas_call(
        body,
        grid=(n // br,),
        in_specs=[
            pl.BlockSpec((2, br, emb), lambda i: (0, i, 0)),
            pl.BlockSpec((br, emb), lambda i: (i, 0)),
            pl.BlockSpec((1, 2, br), lambda i: (i, 0, 0)),
            pl.BlockSpec((1, emb), lambda i: (0, 0)),
        ],
        out_specs=pl.BlockSpec((br, emb), lambda i: (i, 0)),
        out_shape=jax.ShapeDtypeStruct((n, emb), jnp.float32),
    )(agg2, g2, degp3, b2)


def kernel(x, edge_index, edge_weight, W1, b1, W2, b2):
    n, d = x.shape
    e = edge_index.shape[1]
    h = W1.shape[1]
    emb = W2.shape[1]
    src = edge_index[0]
    dst = edge_index[1]

    eps = e // (NC * NS)

    src_r = src.reshape(e // LB, LB)
    dst_r = dst.reshape(e // LB, LB)
    ew_r = edge_weight.reshape(e // LB, LB)
    dst2 = dst.reshape(NC * NS, eps // LB, LB)
    ew2 = edge_weight.reshape(NC * NS, eps)

    degp = _make_deg_kernel(n, e)(dst2, ew2)
    degp3 = degp.reshape(NC, n // 1000, 1000).transpose(1, 0, 2)

    g0 = _tc1(x, degp3)
    agg0 = _make_agg_kernel(n, d, eps)(
        g0, src_r, dst_r, ew_r).reshape(NC, n, d)

    g2 = _tc2(agg0, g0, degp3, W1, b1.reshape(1, h), W2)
    agg2 = _make_agg_kernel(n, emb, eps)(
        g2, src_r, dst_r, ew_r).reshape(NC, n, emb)

    return _tc3(agg2, g2, degp3, b2.reshape(1, emb))
